# Initial kernel scaffold; baseline (speedup 1.0000x reference)
import jax
import jax.numpy as jnp
from jax.experimental import pallas as pl


def kernel(token_table, mask_table, seg_table, region_tokens, region_tokens_mask, segment_ids, region_structures):
    raise NotImplementedError("write your pallas kernel here")


# speedup vs baseline: 5.2343x; 5.2343x over previous
import jax
import jax.numpy as jnp
from jax.experimental import pallas as pl

B = 128
T = 1024
D = 128
VOCAB = 16
N_SEG = 8
MASK_ID = 5
N = B * T
R = 2048
NBLK = N // R


def _emb_block(tok_ref, msk_ref, seg_ref, tok_tab_ref, msk_tab_ref, seg_tab_ref,
               tok_seg_ref, msk_seg_ref, maskpos_ref):
    tok = tok_ref[:, :]
    msk = msk_ref[:, :]
    seg = seg_ref[:, :]
    iota16 = jax.lax.broadcasted_iota(jnp.int32, (R, VOCAB), 1)
    iota8 = jax.lax.broadcasted_iota(jnp.int32, (R, N_SEG), 1)
    oh_tok = ((tok == iota16) & (tok != 0)).astype(jnp.float32)
    oh_msk = ((msk == iota16) & (msk != 0)).astype(jnp.float32)
    oh_seg = ((seg == iota8) & (seg != 0)).astype(jnp.float32)
    tok_emb = jnp.dot(oh_tok, tok_tab_ref[:, :], preferred_element_type=jnp.float32)
    msk_emb = jnp.dot(oh_msk, msk_tab_ref[:, :], preferred_element_type=jnp.float32)
    seg_emb = jnp.dot(oh_seg, seg_tab_ref[:, :], preferred_element_type=jnp.float32)
    tok_seg_ref[:, :D] = tok_emb
    tok_seg_ref[:, D:] = seg_emb
    msk_seg_ref[:, :D] = msk_emb
    msk_seg_ref[:, D:] = seg_emb
    maskpos_ref[:, :] = (msk == MASK_ID).astype(jnp.int32)


def kernel(token_table, mask_table, seg_table, region_tokens, region_tokens_mask, segment_ids, region_structures):
    tok = region_tokens.reshape(N, 1)
    msk = region_tokens_mask.reshape(N, 1)
    seg = segment_ids.reshape(N, 1)

    out_shapes = (
        jax.ShapeDtypeStruct((N, 2 * D), jnp.float32),
        jax.ShapeDtypeStruct((N, 2 * D), jnp.float32),
        jax.ShapeDtypeStruct((N, 1), jnp.int32),
    )
    ids_spec = pl.BlockSpec((R, 1), lambda i: (i, 0))
    tab16_spec = pl.BlockSpec((VOCAB, D), lambda i: (0, 0))
    tab8_spec = pl.BlockSpec((N_SEG, D), lambda i: (0, 0))
    out_spec = pl.BlockSpec((R, 2 * D), lambda i: (i, 0))
    maskpos_spec = pl.BlockSpec((R, 1), lambda i: (i, 0))

    tok_seg, msk_seg, maskpos = pl.pallas_call(
        _emb_block,
        grid=(NBLK,),
        in_specs=[ids_spec, ids_spec, ids_spec, tab16_spec, tab16_spec, tab8_spec],
        out_specs=[out_spec, out_spec, maskpos_spec],
        out_shape=out_shapes,
    )(tok, msk, seg, token_table, mask_table, seg_table)

    tok_seg = tok_seg.reshape(B, T, 2 * D)
    msk_seg = msk_seg.reshape(B, T, 2 * D)
    mask_positions = maskpos.reshape(B, T).astype(jnp.bool_)
    return (tok_seg, msk_seg, region_tokens, region_structures, region_tokens_mask, mask_positions)


# speedup vs baseline: 12.7687x; 2.4394x over previous
import jax
import jax.numpy as jnp
from jax.experimental import pallas as pl

B = 128
T = 1024
D = 128
VOCAB = 16
N_SEG = 8
MASK_ID = 5
N = B * T
R = 2048
RB = R // 128
NBLK = N // R


def _emb_block(tok_ref, msk_ref, seg_ref, tok_tab_ref, msk_tab_ref, seg_tab_ref,
               tok_seg_ref, msk_seg_ref, maskpos_ref):
    tok = tok_ref[...]
    msk = msk_ref[...]
    seg = seg_ref[...]

    tokp = (tok * N_SEG + seg).astype(jnp.float32)
    mskp = (msk * N_SEG + seg).astype(jnp.float32)

    row_id = jax.lax.broadcasted_iota(jnp.int32, (R, RB), 0)
    grp_id = jax.lax.broadcasted_iota(jnp.int32, (R, RB), 1)
    E = (row_id // 128 == grp_id).astype(jnp.float32)
    rr = jax.lax.broadcasted_iota(jnp.int32, (R, 128), 0)
    cc = jax.lax.broadcasted_iota(jnp.int32, (R, 128), 1)
    Dm = (rr % 128 == cc).astype(jnp.float32)
    ones = jnp.ones((128, 1), jnp.float32)

    def flatten(x):
        t1 = jnp.dot(E, x, preferred_element_type=jnp.float32)
        return jnp.dot(t1 * Dm, ones, preferred_element_type=jnp.float32)

    ftok = flatten(tokp)
    fmsk = flatten(mskp)

    ts = jax.lax.broadcasted_iota(jnp.int32, (VOCAB * N_SEG, VOCAB), 0)
    tv = jax.lax.broadcasted_iota(jnp.int32, (VOCAB * N_SEG, VOCAB), 1)
    E16 = ((ts // N_SEG == tv) & (ts // N_SEG != 0)).astype(jnp.float32)
    ss = jax.lax.broadcasted_iota(jnp.int32, (VOCAB * N_SEG, N_SEG), 0)
    sv = jax.lax.broadcasted_iota(jnp.int32, (VOCAB * N_SEG, N_SEG), 1)
    E8 = ((ss % N_SEG == sv) & (ss % N_SEG != 0)).astype(jnp.float32)
    combo_tok = jnp.dot(E16, tok_tab_ref[...], preferred_element_type=jnp.float32)
    combo_msk = jnp.dot(E16, msk_tab_ref[...], preferred_element_type=jnp.float32)
    combo_seg = jnp.dot(E8, seg_tab_ref[...], preferred_element_type=jnp.float32)

    iota128 = jax.lax.broadcasted_iota(jnp.int32, (R, VOCAB * N_SEG), 1)
    oh_tok = (ftok.astype(jnp.int32) == iota128).astype(jnp.float32)
    oh_msk = (fmsk.astype(jnp.int32) == iota128).astype(jnp.float32)

    tok_seg_ref[:, :D] = jnp.dot(oh_tok, combo_tok, preferred_element_type=jnp.float32)
    tok_seg_ref[:, D:] = jnp.dot(oh_tok, combo_seg, preferred_element_type=jnp.float32)
    msk_seg_ref[:, :D] = jnp.dot(oh_msk, combo_msk, preferred_element_type=jnp.float32)
    msk_seg_ref[:, D:] = jnp.dot(oh_msk, combo_seg, preferred_element_type=jnp.float32)
    maskpos_ref[...] = (msk == MASK_ID).astype(jnp.int32)


def kernel(token_table, mask_table, seg_table, region_tokens, region_tokens_mask, segment_ids, region_structures):
    tok = region_tokens.reshape(N // 128, 128)
    msk = region_tokens_mask.reshape(N // 128, 128)
    seg = segment_ids.reshape(N // 128, 128)

    out_shapes = (
        jax.ShapeDtypeStruct((N, 2 * D), jnp.float32),
        jax.ShapeDtypeStruct((N, 2 * D), jnp.float32),
        jax.ShapeDtypeStruct((N // 128, 128), jnp.int32),
    )
    ids_spec = pl.BlockSpec((RB, 128), lambda i: (i, 0))
    tab16_spec = pl.BlockSpec((VOCAB, D), lambda i: (0, 0))
    tab8_spec = pl.BlockSpec((N_SEG, D), lambda i: (0, 0))
    out_spec = pl.BlockSpec((R, 2 * D), lambda i: (i, 0))
    maskpos_spec = pl.BlockSpec((RB, 128), lambda i: (i, 0))

    tok_seg, msk_seg, maskpos = pl.pallas_call(
        _emb_block,
        grid=(NBLK,),
        in_specs=[ids_spec, ids_spec, ids_spec, tab16_spec, tab16_spec, tab8_spec],
        out_specs=[out_spec, out_spec, maskpos_spec],
        out_shape=out_shapes,
    )(tok, msk, seg, token_table, mask_table, seg_table)

    tok_seg = tok_seg.reshape(B, T, 2 * D)
    msk_seg = msk_seg.reshape(B, T, 2 * D)
    mask_positions = maskpos.reshape(B, T).astype(jnp.bool_)
    return (tok_seg, msk_seg, region_tokens, region_structures, region_tokens_mask, mask_positions)


# speedup vs baseline: 15.0794x; 1.1810x over previous
import jax
import jax.numpy as jnp
from jax.experimental import pallas as pl

B = 128
T = 1024
D = 128
VOCAB = 16
N_SEG = 8
MASK_ID = 5
N = B * T
R = 4096
RB = R // 128
NBLK = N // R


def _emb_block(tok_ref, msk_ref, seg_ref, tok_tab_ref, msk_tab_ref, seg_tab_ref,
               tok_seg_ref, msk_seg_ref, maskpos_ref):
    tok = tok_ref[...]
    msk = msk_ref[...]
    seg = seg_ref[...]

    tokp = (tok * N_SEG + seg).astype(jnp.float32)
    mskp = (msk * N_SEG + seg).astype(jnp.float32)

    row_id = jax.lax.broadcasted_iota(jnp.int32, (R, RB), 0)
    grp_id = jax.lax.broadcasted_iota(jnp.int32, (R, RB), 1)
    E = (row_id // 128 == grp_id).astype(jnp.float32)
    rr = jax.lax.broadcasted_iota(jnp.int32, (R, 128), 0)
    cc = jax.lax.broadcasted_iota(jnp.int32, (R, 128), 1)
    Dm = (rr % 128 == cc).astype(jnp.float32)
    ones = jnp.ones((128, 1), jnp.float32)

    def flatten(x):
        t1 = jnp.dot(E, x, preferred_element_type=jnp.float32)
        return jnp.dot(t1 * Dm, ones, preferred_element_type=jnp.float32)

    ftok = flatten(tokp)
    fmsk = flatten(mskp)

    ts = jax.lax.broadcasted_iota(jnp.int32, (VOCAB * N_SEG, VOCAB), 0)
    tv = jax.lax.broadcasted_iota(jnp.int32, (VOCAB * N_SEG, VOCAB), 1)
    E16 = ((ts // N_SEG == tv) & (ts // N_SEG != 0)).astype(jnp.float32)
    ss = jax.lax.broadcasted_iota(jnp.int32, (VOCAB * N_SEG, N_SEG), 0)
    sv = jax.lax.broadcasted_iota(jnp.int32, (VOCAB * N_SEG, N_SEG), 1)
    E8 = ((ss % N_SEG == sv) & (ss % N_SEG != 0)).astype(jnp.float32)
    combo_tok = jnp.dot(E16, tok_tab_ref[...], preferred_element_type=jnp.float32)
    combo_msk = jnp.dot(E16, msk_tab_ref[...], preferred_element_type=jnp.float32)
    combo_seg = jnp.dot(E8, seg_tab_ref[...], preferred_element_type=jnp.float32)

    iota128 = jax.lax.broadcasted_iota(jnp.int32, (R, VOCAB * N_SEG), 1)
    oh_tok = (ftok.astype(jnp.int32) == iota128).astype(jnp.float32)
    oh_msk = (fmsk.astype(jnp.int32) == iota128).astype(jnp.float32)

    tok_seg_ref[:, :D] = jnp.dot(oh_tok, combo_tok, preferred_element_type=jnp.float32)
    tok_seg_ref[:, D:] = jnp.dot(oh_tok, combo_seg, preferred_element_type=jnp.float32)
    msk_seg_ref[:, :D] = jnp.dot(oh_msk, combo_msk, preferred_element_type=jnp.float32)
    msk_seg_ref[:, D:] = jnp.dot(oh_msk, combo_seg, preferred_element_type=jnp.float32)
    maskpos_ref[...] = (msk == MASK_ID).astype(jnp.int32)


def kernel(token_table, mask_table, seg_table, region_tokens, region_tokens_mask, segment_ids, region_structures):
    tok = region_tokens.reshape(N // 128, 128)
    msk = region_tokens_mask.reshape(N // 128, 128)
    seg = segment_ids.reshape(N // 128, 128)

    out_shapes = (
        jax.ShapeDtypeStruct((N, 2 * D), jnp.float32),
        jax.ShapeDtypeStruct((N, 2 * D), jnp.float32),
        jax.ShapeDtypeStruct((N // 128, 128), jnp.int32),
    )
    ids_spec = pl.BlockSpec((RB, 128), lambda i: (i, 0))
    tab16_spec = pl.BlockSpec((VOCAB, D), lambda i: (0, 0))
    tab8_spec = pl.BlockSpec((N_SEG, D), lambda i: (0, 0))
    out_spec = pl.BlockSpec((R, 2 * D), lambda i: (i, 0))
    maskpos_spec = pl.BlockSpec((RB, 128), lambda i: (i, 0))

    tok_seg, msk_seg, maskpos = pl.pallas_call(
        _emb_block,
        grid=(NBLK,),
        in_specs=[ids_spec, ids_spec, ids_spec, tab16_spec, tab16_spec, tab8_spec],
        out_specs=[out_spec, out_spec, maskpos_spec],
        out_shape=out_shapes,
    )(tok, msk, seg, token_table, mask_table, seg_table)

    tok_seg = tok_seg.reshape(B, T, 2 * D)
    msk_seg = msk_seg.reshape(B, T, 2 * D)
    mask_positions = maskpos.reshape(B, T).astype(jnp.bool_)
    return (tok_seg, msk_seg, region_tokens, region_structures, region_tokens_mask, mask_positions)


# speedup vs baseline: 15.3962x; 1.0210x over previous
import jax
import jax.numpy as jnp
from jax.experimental import pallas as pl

B = 128
T = 1024
D = 128
VOCAB = 16
N_SEG = 8
MASK_ID = 5
N = B * T
R = 8192
RB = R // 128
NBLK = N // R


def _emb_block(tok_ref, msk_ref, seg_ref, tok_tab_ref, msk_tab_ref, seg_tab_ref,
               tok_seg_ref, msk_seg_ref, maskpos_ref):
    tok = tok_ref[...]
    msk = msk_ref[...]
    seg = seg_ref[...]

    tokp = (tok * N_SEG + seg).astype(jnp.float32)
    mskp = (msk * N_SEG + seg).astype(jnp.float32)

    row_id = jax.lax.broadcasted_iota(jnp.int32, (R, RB), 0)
    grp_id = jax.lax.broadcasted_iota(jnp.int32, (R, RB), 1)
    E = (row_id // 128 == grp_id).astype(jnp.float32)
    rr = jax.lax.broadcasted_iota(jnp.int32, (R, 128), 0)
    cc = jax.lax.broadcasted_iota(jnp.int32, (R, 128), 1)
    Dm = (rr % 128 == cc).astype(jnp.float32)
    ones = jnp.ones((128, 1), jnp.float32)

    def flatten(x):
        t1 = jnp.dot(E, x, preferred_element_type=jnp.float32)
        return jnp.dot(t1 * Dm, ones, preferred_element_type=jnp.float32)

    ftok = flatten(tokp)
    fmsk = flatten(mskp)

    ts = jax.lax.broadcasted_iota(jnp.int32, (VOCAB * N_SEG, VOCAB), 0)
    tv = jax.lax.broadcasted_iota(jnp.int32, (VOCAB * N_SEG, VOCAB), 1)
    E16 = ((ts // N_SEG == tv) & (ts // N_SEG != 0)).astype(jnp.float32)
    ss = jax.lax.broadcasted_iota(jnp.int32, (VOCAB * N_SEG, N_SEG), 0)
    sv = jax.lax.broadcasted_iota(jnp.int32, (VOCAB * N_SEG, N_SEG), 1)
    E8 = ((ss % N_SEG == sv) & (ss % N_SEG != 0)).astype(jnp.float32)
    combo_tok = jnp.dot(E16, tok_tab_ref[...], preferred_element_type=jnp.float32)
    combo_msk = jnp.dot(E16, msk_tab_ref[...], preferred_element_type=jnp.float32)
    combo_seg = jnp.dot(E8, seg_tab_ref[...], preferred_element_type=jnp.float32)

    iota128 = jax.lax.broadcasted_iota(jnp.int32, (R, VOCAB * N_SEG), 1)
    oh_tok = (ftok.astype(jnp.int32) == iota128).astype(jnp.float32)
    oh_msk = (fmsk.astype(jnp.int32) == iota128).astype(jnp.float32)

    tok_seg_ref[:, :D] = jnp.dot(oh_tok, combo_tok, preferred_element_type=jnp.float32)
    tok_seg_ref[:, D:] = jnp.dot(oh_tok, combo_seg, preferred_element_type=jnp.float32)
    msk_seg_ref[:, :D] = jnp.dot(oh_msk, combo_msk, preferred_element_type=jnp.float32)
    msk_seg_ref[:, D:] = jnp.dot(oh_msk, combo_seg, preferred_element_type=jnp.float32)
    maskpos_ref[...] = (msk == MASK_ID).astype(jnp.int32)


def kernel(token_table, mask_table, seg_table, region_tokens, region_tokens_mask, segment_ids, region_structures):
    tok = region_tokens.reshape(N // 128, 128)
    msk = region_tokens_mask.reshape(N // 128, 128)
    seg = segment_ids.reshape(N // 128, 128)

    out_shapes = (
        jax.ShapeDtypeStruct((N, 2 * D), jnp.float32),
        jax.ShapeDtypeStruct((N, 2 * D), jnp.float32),
        jax.ShapeDtypeStruct((N // 128, 128), jnp.int32),
    )
    ids_spec = pl.BlockSpec((RB, 128), lambda i: (i, 0))
    tab16_spec = pl.BlockSpec((VOCAB, D), lambda i: (0, 0))
    tab8_spec = pl.BlockSpec((N_SEG, D), lambda i: (0, 0))
    out_spec = pl.BlockSpec((R, 2 * D), lambda i: (i, 0))
    maskpos_spec = pl.BlockSpec((RB, 128), lambda i: (i, 0))

    tok_seg, msk_seg, maskpos = pl.pallas_call(
        _emb_block,
        grid=(NBLK,),
        in_specs=[ids_spec, ids_spec, ids_spec, tab16_spec, tab16_spec, tab8_spec],
        out_specs=[out_spec, out_spec, maskpos_spec],
        out_shape=out_shapes,
    )(tok, msk, seg, token_table, mask_table, seg_table)

    tok_seg = tok_seg.reshape(B, T, 2 * D)
    msk_seg = msk_seg.reshape(B, T, 2 * D)
    mask_positions = maskpos.reshape(B, T).astype(jnp.bool_)
    return (tok_seg, msk_seg, region_tokens, region_structures, region_tokens_mask, mask_positions)
